# baseline (speedup 1.0000x reference)
import functools

import jax
import jax.numpy as jnp
from jax import lax
from jax.experimental import pallas as pl
from jax.experimental.pallas import tpu as pltpu
from jax.experimental.pallas import tpu_sc as plsc

NC, NS = 2, 16
NW = NC * NS
D = 64
B = 4096 * 200
BPW = B // NW
GR = 512
CH = 512
NG = CH // GR
NCHUNK = BPW // CH
NBUF = 2

_mesh = plsc.VectorSubcoreMesh(core_axis_name="c", subcore_axis_name="s")


@functools.partial(
    pl.kernel,
    out_type=jax.ShapeDtypeStruct((B, D), jnp.float32),
    mesh=_mesh,
    compiler_params=pltpu.CompilerParams(use_tc_tiling_on_sc=False),
    scratch_types=[
        pltpu.VMEM((NCHUNK * NG, GR), jnp.int32),
        pltpu.VMEM((NBUF, CH, D), jnp.float32),
        pltpu.SemaphoreType.DMA,
        pltpu.SemaphoreType.DMA,
        pltpu.SemaphoreType.DMA,
    ],
)
def _embed(table, xidx, out, idx_v, rows_v, gsem, osem0, osem1):
    wid = lax.axis_index("s") * NC + lax.axis_index("c")
    base = wid * BPW
    pltpu.sync_copy(xidx.at[wid], idx_v)
    osems = (osem0, osem1)

    def gather_descs(c, b):
        return [
            pltpu.make_async_copy(
                table.at[idx_v.at[c * NG + j]],
                rows_v.at[b, pl.ds(j * GR, GR)],
                gsem,
            )
            for j in range(NG)
        ]

    def out_desc(c, b):
        off = pl.multiple_of(base + c * CH, CH)
        return pltpu.make_async_copy(
            rows_v.at[b], out.at[pl.ds(off, CH)], osems[b]
        )

    for b in range(NBUF):
        for d in gather_descs(b, b):
            d.start()

    def group(g, carry):
        for b in range(NBUF):
            c = g * NBUF + b
            for d in gather_descs(c, b):
                d.wait()
            od = out_desc(c, b)
            od.start()
            nxt = c + NBUF

            @pl.when(nxt < NCHUNK)
            def _():
                od.wait()
                for d in gather_descs(nxt, b):
                    d.start()

        return carry

    lax.fori_loop(0, NCHUNK // NBUF, group, 0)

    for b in range(NBUF):
        out_desc(NCHUNK - NBUF + b, b).wait()


def kernel(X, table):
    xidx = X.reshape(NW, NCHUNK * NG, GR)
    out = _embed(table, xidx)
    return out.reshape(X.shape[0], X.shape[1], D)


# speedup vs baseline: 1.0025x; 1.0025x over previous
import functools

import jax
import jax.numpy as jnp
from jax import lax
from jax.experimental import pallas as pl
from jax.experimental.pallas import tpu as pltpu
from jax.experimental.pallas import tpu_sc as plsc

NC, NS = 2, 16
NW = NC * NS
D = 64
XR = 4096
HIST = 200
XPW = XR // NW
RPC = 2
NCHUNK = XPW // RPC
NBUF = 2

_mesh = plsc.VectorSubcoreMesh(core_axis_name="c", subcore_axis_name="s")


@functools.partial(
    pl.kernel,
    out_type=jax.ShapeDtypeStruct((XR, HIST, D), jnp.float32),
    mesh=_mesh,
    compiler_params=pltpu.CompilerParams(use_tc_tiling_on_sc=False),
    scratch_types=[
        pltpu.VMEM((XPW, HIST), jnp.int32),
        pltpu.VMEM((NBUF, RPC, HIST, D), jnp.float32),
        pltpu.SemaphoreType.DMA,
        pltpu.SemaphoreType.DMA,
        pltpu.SemaphoreType.DMA,
    ],
)
def _embed(table, xidx, out, idx_v, rows_v, gsem, osem0, osem1):
    wid = lax.axis_index("s") * NC + lax.axis_index("c")
    xbase = wid * XPW
    pltpu.sync_copy(xidx.at[pl.ds(xbase, XPW)], idx_v)
    osems = (osem0, osem1)

    def gather_descs(c, b):
        return [
            pltpu.make_async_copy(
                table.at[idx_v.at[RPC * c + j]],
                rows_v.at[b, j],
                gsem,
            )
            for j in range(RPC)
        ]

    def out_desc(c, b):
        off = pl.multiple_of(xbase + RPC * c, RPC)
        return pltpu.make_async_copy(
            rows_v.at[b], out.at[pl.ds(off, RPC)], osems[b]
        )

    for b in range(NBUF):
        for d in gather_descs(b, b):
            d.start()

    def group(g, carry):
        for b in range(NBUF):
            c = g * NBUF + b
            for d in gather_descs(c, b):
                d.wait()
            od = out_desc(c, b)
            od.start()
            nxt = c + NBUF

            @pl.when(nxt < NCHUNK)
            def _():
                od.wait()
                for d in gather_descs(nxt, b):
                    d.start()

        return carry

    lax.fori_loop(0, NCHUNK // NBUF, group, 0)

    for b in range(NBUF):
        out_desc(NCHUNK - NBUF + b, b).wait()


def kernel(X, table):
    return _embed(table, X)


# speedup vs baseline: 1.2244x; 1.2213x over previous
import functools

import jax
import jax.numpy as jnp
from jax import lax
from jax.experimental import pallas as pl
from jax.experimental.pallas import tpu as pltpu
from jax.experimental.pallas import tpu_sc as plsc

NC, NS = 2, 16
NW = NC * NS
D = 64
DP = 128
B = 4096 * 200
BPW = B // NW
CH = 256
NCHUNK = BPW // CH
NBUF = 2

_mesh = plsc.VectorSubcoreMesh(core_axis_name="c", subcore_axis_name="s")


@functools.partial(
    pl.kernel,
    out_type=jax.ShapeDtypeStruct((B, DP), jnp.float32),
    mesh=_mesh,
    compiler_params=pltpu.CompilerParams(use_tc_tiling_on_sc=True),
    scratch_types=[
        pltpu.VMEM((BPW,), jnp.int32),
        pltpu.VMEM((NBUF, CH, DP), jnp.float32),
        pltpu.SemaphoreType.DMA,
        pltpu.SemaphoreType.DMA,
        pltpu.SemaphoreType.DMA,
    ],
)
def _embed(table, xflat, out, idx_v, rows_v, gsem, osem0, osem1):
    wid = lax.axis_index("s") * NC + lax.axis_index("c")
    base = wid * BPW
    pltpu.sync_copy(xflat.at[pl.ds(base, BPW)], idx_v)
    osems = (osem0, osem1)

    def gather_desc(c, b):
        off = pl.multiple_of(c * CH, CH)
        return pltpu.make_async_copy(
            table.at[idx_v.at[pl.ds(off, CH)]], rows_v.at[b], gsem
        )

    def out_desc(c, b):
        off = pl.multiple_of(base + c * CH, CH)
        return pltpu.make_async_copy(
            rows_v.at[b], out.at[pl.ds(off, CH)], osems[b]
        )

    for b in range(NBUF):
        gather_desc(b, b).start()

    def group(g, carry):
        for b in range(NBUF):
            c = g * NBUF + b
            gather_desc(c, b).wait()
            od = out_desc(c, b)
            od.start()
            nxt = c + NBUF

            @pl.when(nxt < NCHUNK)
            def _():
                od.wait()
                gather_desc(nxt, b).start()

        return carry

    lax.fori_loop(0, NCHUNK // NBUF, group, 0)

    for b in range(NBUF):
        out_desc(NCHUNK - NBUF + b, b).wait()


def kernel(X, table):
    tablep = jnp.pad(table, ((0, 0), (0, DP - D)))
    out = _embed(tablep, X.reshape(-1))
    return out[:, :D].reshape(X.shape[0], X.shape[1], D)


# speedup vs baseline: 1.4318x; 1.1694x over previous
import functools

import jax
import jax.numpy as jnp
from jax import lax
from jax.experimental import pallas as pl
from jax.experimental.pallas import tpu as pltpu
from jax.experimental.pallas import tpu_sc as plsc

NC, NS = 2, 16
NW = NC * NS
D = 64
DP = 128
B = 4096 * 200
BPW = B // NW
CH = 256
NCHUNK = BPW // CH
NBUF = 2

_mesh = plsc.VectorSubcoreMesh(core_axis_name="c", subcore_axis_name="s")


@functools.partial(
    pl.kernel,
    out_type=jax.ShapeDtypeStruct((B, DP), jnp.float32),
    mesh=_mesh,
    compiler_params=pltpu.CompilerParams(use_tc_tiling_on_sc=False),
    scratch_types=[
        pltpu.VMEM((BPW,), jnp.int32),
        pltpu.VMEM((NBUF, CH, D), jnp.float32),
        pltpu.SemaphoreType.DMA,
        pltpu.SemaphoreType.DMA,
        pltpu.SemaphoreType.DMA,
    ],
)
def _embed(t64, xflat2, out, idx_v, rows_v, gsem, osem0, osem1):
    wid = lax.axis_index("s") * NC + lax.axis_index("c")
    base = wid * BPW
    pltpu.sync_copy(xflat2.at[pl.ds(base, BPW)], idx_v)
    osems = (osem0, osem1)

    def gather_desc(c, b):
        off = pl.multiple_of(c * CH, CH)
        return pltpu.make_async_copy(
            t64.at[idx_v.at[pl.ds(off, CH)]], rows_v.at[b], gsem
        )

    def out_desc(c, b):
        off = pl.multiple_of(base + c * CH, CH)
        return pltpu.make_async_copy(
            rows_v.at[b],
            out.at[pl.ds(off, CH), pl.ds(0, D)],
            osems[b],
        )

    for b in range(NBUF):
        gather_desc(b, b).start()

    def group(g, carry):
        for b in range(NBUF):
            c = g * NBUF + b
            gather_desc(c, b).wait()
            od = out_desc(c, b)
            od.start()
            nxt = c + NBUF

            @pl.when(nxt < NCHUNK)
            def _():
                od.wait()
                gather_desc(nxt, b).start()

        return carry

    lax.fori_loop(0, NCHUNK // NBUF, group, 0)

    for b in range(NBUF):
        out_desc(NCHUNK - NBUF + b, b).wait()


def kernel(X, table):
    tablep = jnp.pad(table, ((0, 0), (0, DP - D)))
    t64 = tablep.reshape(2 * table.shape[0], D)
    xflat2 = (X * 2).reshape(-1)
    out = _embed(t64, xflat2)
    return out[:, :D].reshape(X.shape[0], X.shape[1], D)
